# Initial kernel scaffold; baseline (speedup 1.0000x reference)
import jax
import jax.numpy as jnp
from jax.experimental import pallas as pl


def kernel(atom_ids, hybrid_ids, node_continuous, atom_table, hybrid_table, W_cont, b_cont, W_fuse, b_fuse, W_out, b_out):
    raise NotImplementedError("write your pallas kernel here")


# speedup vs baseline: 4.0356x; 4.0356x over previous
import functools

import jax
import jax.numpy as jnp
from jax import lax
from jax.experimental import pallas as pl
from jax.experimental.pallas import tpu as pltpu
from jax.experimental.pallas import tpu_sc as plsc

B, N = 4096, 200
ROWS = B * N
AV, HV = 1000, 8
AD, HD, CD, HID = 32, 16, 16, 64
CF = 8
TV = AV * HV

NC, NS, L = 2, 16, 16
NW = NC * NS
RPW = ROWS // NW
CH = 128
NCH = RPW // CH
IDROWS = ROWS // 128
IDR_PW = IDROWS // NW


def _fold_body(atom_ref, hyb_ref, wc_ref, bc_ref, wf_ref, bf_ref,
               t3_ref, w2_ref):
    wf = wf_ref[...]
    a2 = jnp.dot(atom_ref[...], wf[0:AD, :],
                 preferred_element_type=jnp.float32)
    h2 = jnp.dot(hyb_ref[...], wf[AD:AD + HD, :],
                 preferred_element_type=jnp.float32)
    c2 = jnp.dot(bc_ref[...], wf[AD + HD:, :],
                 preferred_element_type=jnp.float32) + bf_ref[...]
    t3_ref[...] = a2[:, None, :] + (h2 + c2)[None, :, :]
    w2_ref[...] = jnp.dot(wc_ref[...], wf[AD + HD:, :],
                          preferred_element_type=jnp.float32)


def _fold(atom_table, hybrid_table, W_cont, b_cont, W_fuse, b_fuse):
    return pl.pallas_call(
        _fold_body,
        out_shape=(
            jax.ShapeDtypeStruct((AV, HV, HID), jnp.float32),
            jax.ShapeDtypeStruct((CF, HID), jnp.float32),
        ),
    )(atom_table, hybrid_table, W_cont, b_cont.reshape(1, CD),
      W_fuse, b_fuse.reshape(1, HID))


def _sc_gather_body(aid_hbm, hid_hbm, t_hbm, out_hbm,
                    cid_v, hid_v, rbuf, gsem, ssem):
    wid = lax.axis_index("s") * NC + lax.axis_index("c")
    base_ids = wid * IDR_PW
    base_out = wid * RPW

    pltpu.sync_copy(aid_hbm.at[pl.ds(base_ids, IDR_PW)], cid_v)
    pltpu.sync_copy(hid_hbm.at[pl.ds(base_ids, IDR_PW)], hid_v)

    def cid_body(j, carry):
        for i in range(128 // L):
            sl = pl.ds(i * L, L)
            cid_v[j, sl] = cid_v[j, sl] * HV + hid_v[j, sl]
        return carry
    lax.fori_loop(0, IDR_PW, cid_body, 0)

    def g_body(j, carry):
        pltpu.async_copy(t_hbm.at[cid_v.at[j]], rbuf, gsem).wait()
        pltpu.async_copy(rbuf, out_hbm.at[pl.ds(base_out + j * CH, CH)],
                         ssem).wait()
        return carry
    lax.fori_loop(0, NCH, g_body, 0)


def _sc_gather(aid2, hid2, T):
    mesh = plsc.VectorSubcoreMesh(core_axis_name="c", subcore_axis_name="s",
                                  num_cores=NC, num_subcores=NS)
    fn = pl.kernel(
        _sc_gather_body,
        out_type=jax.ShapeDtypeStruct((ROWS, HID), jnp.float32),
        mesh=mesh,
        compiler_params=pltpu.CompilerParams(use_tc_tiling_on_sc=False),
        scratch_types=[
            pltpu.VMEM((IDR_PW, 128), jnp.int32),
            pltpu.VMEM((IDR_PW, 128), jnp.int32),
            pltpu.VMEM((CH, HID), jnp.float32),
            pltpu.SemaphoreType.DMA,
            pltpu.SemaphoreType.DMA,
        ],
    )
    return fn(aid2, hid2, T)


RB = 2048


def _mlp_body(g_ref, x_ref, w2_ref, wo_ref, bo_ref, out_ref):
    h = g_ref[...] + jnp.dot(x_ref[...], w2_ref[...],
                             preferred_element_type=jnp.float32)
    h = jax.nn.gelu(h)
    out_ref[...] = jnp.dot(h, wo_ref[...],
                           preferred_element_type=jnp.float32) + bo_ref[...]


def _mlp(G, X, W2, W_out, b_out):
    grid = (ROWS // RB,)
    return pl.pallas_call(
        _mlp_body,
        grid=grid,
        in_specs=[
            pl.BlockSpec((RB, HID), lambda i: (i, 0)),
            pl.BlockSpec((RB, CF), lambda i: (i, 0)),
            pl.BlockSpec((CF, HID), lambda i: (0, 0)),
            pl.BlockSpec((HID, HID), lambda i: (0, 0)),
            pl.BlockSpec((1, HID), lambda i: (0, 0)),
        ],
        out_specs=pl.BlockSpec((RB, HID), lambda i: (i, 0)),
        out_shape=jax.ShapeDtypeStruct((ROWS, HID), jnp.float32),
    )(G, X, W2, W_out, b_out.reshape(1, HID))


def kernel(atom_ids, hybrid_ids, node_continuous, atom_table, hybrid_table,
           W_cont, b_cont, W_fuse, b_fuse, W_out, b_out):
    T3, W2 = _fold(atom_table, hybrid_table, W_cont, b_cont, W_fuse, b_fuse)
    T = T3.reshape(TV, HID)
    aid2 = atom_ids.reshape(IDROWS, 128)
    hid2 = hybrid_ids.reshape(IDROWS, 128)
    G = _sc_gather(aid2, hid2, T)
    X = node_continuous.reshape(ROWS, CF)
    out = _mlp(G, X, W2, W_out, b_out)
    return out.reshape(B, N, HID)
